# baseline (speedup 1.0000x reference)
import functools

import jax
import jax.numpy as jnp
from jax import lax
from jax.experimental import pallas as pl
from jax.experimental.pallas import tpu as pltpu
from jax.experimental.pallas import tpu_sc as plsc

N = 100000
N_SPECIES = 119
L = 16
NS = 16
P = 6272
H = P // 2

_mesh = plsc.VectorSubcoreMesh(
    core_axis_name="c", subcore_axis_name="s", num_cores=1)


@functools.partial(
    pl.kernel,
    mesh=_mesh,
    out_type=jax.ShapeDtypeStruct((N,), jnp.float32),
    scratch_types=[
        pltpu.VMEM((P,), jnp.float32),
        pltpu.VMEM((P,), jnp.int32),
        pltpu.VMEM((P,), jnp.float32),
        pltpu.VMEM((N_SPECIES,), jnp.float32),
        pltpu.SemaphoreType.DMA,
        pltpu.SemaphoreType.DMA,
        pltpu.SemaphoreType.DMA,
    ],
    compiler_params=pltpu.CompilerParams(
        needs_layout_passes=False,
        disable_bounds_checks=True,
        disable_semaphore_checks=True,
        skip_device_barrier=True,
    ),
)
def _per_species_offset(x_hbm, z_hbm, off_hbm, out_hbm, x_v, z_v, o_v, tab_v,
                        sem0, sem1, semo):
    wid = lax.axis_index("s")
    base = jnp.minimum(wid * P, N - P)

    tab_cp = pltpu.async_copy(off_hbm, tab_v, sem0)
    x0_cp = pltpu.async_copy(x_hbm.at[pl.ds(base, H)],
                             x_v.at[pl.ds(0, H)], sem0)
    z0_cp = pltpu.async_copy(z_hbm.at[pl.ds(base, H)],
                             z_v.at[pl.ds(0, H)], sem0)
    x1_cp = pltpu.async_copy(x_hbm.at[pl.ds(base + H, H)],
                             x_v.at[pl.ds(H, H)], sem1)
    z1_cp = pltpu.async_copy(z_hbm.at[pl.ds(base + H, H)],
                             z_v.at[pl.ds(H, H)], sem1)
    tab_cp.wait()
    x0_cp.wait()
    z0_cp.wait()

    @plsc.parallel_loop(0, H, L, unroll=4)
    def _(s):
        o_v[pl.ds(s, L)] = x_v[pl.ds(s, L)] + plsc.load_gather(
            tab_v, [z_v[pl.ds(s, L)]])

    o0_cp = pltpu.async_copy(o_v.at[pl.ds(0, H)],
                             out_hbm.at[pl.ds(base, H)], semo)
    x1_cp.wait()
    z1_cp.wait()

    @plsc.parallel_loop(H, P, L, unroll=4)
    def _(s):
        o_v[pl.ds(s, L)] = x_v[pl.ds(s, L)] + plsc.load_gather(
            tab_v, [z_v[pl.ds(s, L)]])

    pltpu.sync_copy(o_v.at[pl.ds(H, H)], out_hbm.at[pl.ds(base + H, H)])
    o0_cp.wait()


def kernel(x, Z, offsets):
    return _per_species_offset(x, Z.astype(jnp.int32), offsets)


# speedup vs baseline: 1.0066x; 1.0066x over previous
import functools

import jax
import jax.numpy as jnp
from jax import lax
from jax.experimental import pallas as pl
from jax.experimental.pallas import tpu as pltpu
from jax.experimental.pallas import tpu_sc as plsc

N = 100000
N_SPECIES = 119
L = 16
NS = 16
P = 6272

_mesh = plsc.VectorSubcoreMesh(
    core_axis_name="c", subcore_axis_name="s", num_cores=1)


@functools.partial(
    pl.kernel,
    mesh=_mesh,
    out_type=jax.ShapeDtypeStruct((N,), jnp.float32),
    scratch_types=[
        pltpu.VMEM((P,), jnp.float32),
        pltpu.VMEM((P,), jnp.int32),
        pltpu.VMEM((P,), jnp.float32),
        pltpu.VMEM((N_SPECIES,), jnp.float32),
        pltpu.SemaphoreType.DMA,
    ],
    compiler_params=pltpu.CompilerParams(
        needs_layout_passes=False,
        disable_bounds_checks=True,
        disable_semaphore_checks=True,
        skip_device_barrier=True,
    ),
)
def _per_species_offset(x_hbm, z_hbm, off_hbm, out_hbm, x_v, z_v, o_v, tab_v,
                        sem):
    wid = lax.axis_index("s")
    base = jnp.minimum(wid * P, N - P)

    tab_cp = pltpu.async_copy(off_hbm, tab_v, sem)
    x_cp = pltpu.async_copy(x_hbm.at[pl.ds(base, P)], x_v, sem)
    z_cp = pltpu.async_copy(z_hbm.at[pl.ds(base, P)], z_v, sem)
    tab_cp.wait()
    x_cp.wait()
    z_cp.wait()

    @plsc.parallel_loop(0, P, L, unroll=4)
    def _(s):
        o_v[pl.ds(s, L)] = x_v[pl.ds(s, L)] + plsc.load_gather(
            tab_v, [z_v[pl.ds(s, L)]])

    pltpu.sync_copy(o_v, out_hbm.at[pl.ds(base, P)])


def kernel(x, Z, offsets):
    return _per_species_offset(x, Z.astype(jnp.int32), offsets)
